# baseline (speedup 1.0000x reference)
import dataclasses
import functools

import jax
import jax.numpy as jnp
from jax import lax
from jax.experimental import pallas as pl
from jax.experimental.pallas import tpu as pltpu
from jax.experimental.pallas import tpu_sc as plsc

N_NODES = 10000
H_DIM = 128
N_EDGES = 320000
NUM_RELS = 16

NW = 32
CHUNK = 80
G = CHUNK // 16
NCH = N_EDGES // CHUNK // NW
PKW = 4 * CHUNK
NPAD = 10240
ROWS_PER_SUB = NPAD // 16
ZR = 32


def _sc_layer(h, pk, W):
    mesh = plsc.VectorSubcoreMesh(core_axis_name="c", subcore_axis_name="s")
    cp = pltpu.CompilerParams()
    if "needs_layout_passes" in pltpu.CompilerParams.__dataclass_fields__:
        cp = dataclasses.replace(cp, needs_layout_passes=False)

    @functools.partial(
        pl.kernel,
        compiler_params=cp,
        out_type=jax.ShapeDtypeStruct((2, NPAD, H_DIM), jnp.float32),
        mesh=mesh,
        scratch_types=[
            pltpu.VMEM((3 * PKW,), jnp.int32),
            pltpu.VMEM((3, CHUNK, H_DIM), jnp.float32),
            pltpu.VMEM((2, 1, CHUNK), jnp.int32),
            pltpu.VMEM((NUM_RELS * H_DIM // 2,), jnp.int32),
            pltpu.VMEM((NPAD,), jnp.int32),
            pltpu.VMEM((ZR, H_DIM), jnp.float32),
            pltpu.VMEM_SHARED((NPAD, H_DIM), jnp.float32),
            pltpu.SemaphoreType.DMA,
            pltpu.SemaphoreType.DMA,
            pltpu.SemaphoreType.DMA,
        ],
    )
    def k(h_hbm, pk_hbm, w_hbm, out_hbm,
          pb, rows, mi, wv, tag, zv, acc, msem, gsem, ssem):
        c = lax.axis_index("c")
        s = lax.axis_index("s")
        wid = c * 16 + s
        first = wid * NCH

        pltpu.sync_copy(w_hbm, wv)

        @pl.loop(0, ZR)
        def _(r):
            for j in range(H_DIM // 16):
                zv[r, pl.ds(j * 16, 16)] = jnp.zeros((16,), jnp.float32)

        @pl.loop(0, ROWS_PER_SUB // ZR)
        def _(z):
            pltpu.sync_copy(
                zv, acc.at[pl.ds(s * ROWS_PER_SUB + z * ZR, ZR), :])

        plsc.subcore_barrier()

        lane = jnp.arange(16, dtype=jnp.int32)

        pltpu.async_copy(pk_hbm.at[pl.ds(first * PKW, PKW)],
                         pb.at[pl.ds(0, PKW)], msem).wait()
        pltpu.async_copy(h_hbm.at[pb.at[pl.ds(0, CHUNK)]],
                         rows.at[0], gsem)
        pltpu.async_copy(pk_hbm.at[pl.ds((first + 1) * PKW, PKW)],
                         pb.at[pl.ds(PKW, PKW)], msem)

        @pl.loop(0, NCH)
        def _(i):
            b3 = lax.rem(i, 3)
            b3n = lax.rem(i + 1, 3)
            b3nn = lax.rem(i + 2, 3)
            b2 = lax.rem(i, 2)

            @pl.when(i >= 2)
            def _():
                pltpu.make_async_copy(
                    rows.at[b3n], acc.at[mi.at[b2, 0]], ssem).wait()

            @pl.when(i + 1 < NCH)
            def _():
                pltpu.make_async_copy(
                    pk_hbm.at[pl.ds((first + i + 1) * PKW, PKW)],
                    pb.at[pl.ds(b3n * PKW, PKW)], msem).wait()
                pltpu.async_copy(h_hbm.at[pb.at[pl.ds(b3n * PKW, CHUNK)]],
                                 rows.at[b3n], gsem)

            @pl.when(i + 2 < NCH)
            def _():
                pltpu.async_copy(
                    pk_hbm.at[pl.ds((first + i + 2) * PKW, PKW)],
                    pb.at[pl.ds(b3nn * PKW, PKW)], msem)

            pltpu.make_async_copy(h_hbm.at[pb.at[pl.ds(b3 * PKW, CHUNK)]],
                                  rows.at[b3], gsem).wait()

            idbase = i * CHUNK

            for g in range(G):
                et16 = pb[pl.ds(b3 * PKW + 2 * CHUNK + g * 16, 16)]
                nb16 = pb[pl.ds(b3 * PKW + 3 * CHUNK + g * 16, 16)]
                nm16 = plsc.bitcast(nb16, jnp.float32)
                def _edge_loads(el):
                    rel = et16[el]
                    r = g * 16 + el
                    wvals = []
                    for j in range(H_DIM // 32):
                        v = wv[pl.ds(rel * (H_DIM // 2) + j * 16, 16)]
                        wlo = plsc.bitcast(v << 16, jnp.float32)
                        whi = plsc.bitcast(v & jnp.int32(-65536), jnp.float32)
                        wvals += [wlo, whi]
                    rvals = [rows[b3, r, pl.ds(j * 16, 16)]
                             for j in range(H_DIM // 16)]
                    return wvals, rvals

                cur = _edge_loads(0)
                for el in range(16):
                    nxt = _edge_loads(el + 1) if el < 15 else None
                    sc = nm16[el]
                    r = g * 16 + el
                    wvals, rvals = cur
                    for j in range(H_DIM // 16):
                        rows[b3, r, pl.ds(j * 16, 16)] = (
                            rvals[j] * (wvals[j] * sc))
                    cur = nxt
                dst16 = pb[pl.ds(b3 * PKW + CHUNK + g * 16, 16)]
                plsc.store_scatter(tag, [dst16], idbase + g * 16 + lane)

            for g in range(G):
                dst16 = pb[pl.ds(b3 * PKW + CHUNK + g * 16, 16)]
                ids16 = idbase + g * 16 + lane
                rb16 = plsc.load_gather(tag, [dst16])
                losei = (rb16 != ids16).astype(jnp.int32)
                mi[b2, 0, pl.ds(g * 16, 16)] = jnp.where(
                    losei == 1, N_NODES + lane, dst16)

                @pl.when(jnp.sum(losei) > 0)
                def _():
                    for el in range(16):
                        @pl.when(losei[el] == 1)
                        def _():
                            w = rb16[el] - idbase
                            r = g * 16 + el
                            for j in range(H_DIM // 16):
                                sl = pl.ds(j * 16, 16)
                                rows[b3, w, sl] = (rows[b3, w, sl]
                                                   + rows[b3, r, sl])

            pltpu.async_copy(rows.at[b3], acc.at[mi.at[b2, 0]], ssem,
                             add=True)

        pltpu.make_async_copy(rows.at[0], acc.at[mi.at[0, 0]], ssem).wait()
        pltpu.make_async_copy(rows.at[0], acc.at[mi.at[0, 0]], ssem).wait()

        plsc.subcore_barrier()

        pltpu.sync_copy(
            acc.at[pl.ds(s * ROWS_PER_SUB, ROWS_PER_SUB), :],
            out_hbm.at[c, pl.ds(s * ROWS_PER_SUB, ROWS_PER_SUB), :])

    return k(h, pk, W)


def _combine(p, h, loop_w, b, relu):
    R = 1000

    def body(p_ref, h_ref, w_ref, b_ref, o_ref):
        acc = (p_ref[0] + p_ref[1]
               + jnp.dot(h_ref[...], w_ref[...],
                         preferred_element_type=jnp.float32)
               + b_ref[...])
        if relu:
            acc = jnp.maximum(acc, 0.0)
        o_ref[...] = acc

    return pl.pallas_call(
        body,
        grid=(N_NODES // R,),
        in_specs=[
            pl.BlockSpec((2, R, H_DIM), lambda i: (0, i, 0)),
            pl.BlockSpec((R, H_DIM), lambda i: (i, 0)),
            pl.BlockSpec((H_DIM, H_DIM), lambda i: (0, 0)),
            pl.BlockSpec((1, H_DIM), lambda i: (0, 0)),
        ],
        out_specs=pl.BlockSpec((R, H_DIM), lambda i: (i, 0)),
        out_shape=jax.ShapeDtypeStruct((N_NODES, H_DIM), jnp.float32),
    )(p, h, loop_w, b.reshape(1, H_DIM))


def _pack_w(W):
    bits = lax.bitcast_convert_type(W.astype(jnp.bfloat16),
                                    jnp.uint16).astype(jnp.uint32)
    br = bits.reshape(NUM_RELS, H_DIM // 32, 2, 16)
    packed = (br[:, :, 1, :] << 16) | br[:, :, 0, :]
    return packed.reshape(-1).astype(jnp.int32)


def kernel(x, edge_index, etype, norm, W1, loop1, b1, W2, loop2, b2):
    src = edge_index[0].astype(jnp.int32)
    dst = edge_index[1].astype(jnp.int32)
    et = etype.astype(jnp.int32)
    nbits = lax.bitcast_convert_type(norm.reshape(-1), jnp.int32)
    pk = jnp.stack([src.reshape(-1, CHUNK), dst.reshape(-1, CHUNK),
                    et.reshape(-1, CHUNK), nbits.reshape(-1, CHUNK)],
                   axis=1).reshape(-1)

    p1 = _sc_layer(x, pk, _pack_w(W1))
    h1 = _combine(p1, x, loop1, b1, relu=True)
    p2 = _sc_layer(h1, pk, _pack_w(W2))
    return _combine(p2, h1, loop2, b2, relu=False)


# speedup vs baseline: 2.0670x; 2.0670x over previous
import dataclasses
import functools

import jax
import jax.numpy as jnp
from jax import lax
from jax.experimental import pallas as pl
from jax.experimental.pallas import tpu as pltpu
from jax.experimental.pallas import tpu_sc as plsc

N_NODES = 10000
H_DIM = 128
N_EDGES = 320000
NUM_RELS = 16

NW = 32
CHUNK = 80
G = CHUNK // 16
NCH = N_EDGES // CHUNK // NW
PKW = 4 * CHUNK
NPAD = 10240
ROWS_PER_SUB = NPAD // 16
ZR = 32


def _sc_layer(h, pk, W):
    mesh = plsc.VectorSubcoreMesh(core_axis_name="c", subcore_axis_name="s")
    cp = pltpu.CompilerParams()
    if "needs_layout_passes" in pltpu.CompilerParams.__dataclass_fields__:
        cp = dataclasses.replace(cp, needs_layout_passes=False)

    @functools.partial(
        pl.kernel,
        compiler_params=cp,
        out_type=jax.ShapeDtypeStruct((2, NPAD, H_DIM), jnp.float32),
        mesh=mesh,
        scratch_types=[
            pltpu.VMEM((3 * PKW,), jnp.int32),
            pltpu.VMEM((3, CHUNK, H_DIM), jnp.float32),
            pltpu.VMEM((2, 1, CHUNK), jnp.int32),
            pltpu.VMEM((NUM_RELS * H_DIM // 2,), jnp.int32),
            pltpu.VMEM((NPAD,), jnp.int32),
            pltpu.VMEM((ZR, H_DIM), jnp.float32),
            pltpu.VMEM_SHARED((NPAD, H_DIM), jnp.float32),
            pltpu.SemaphoreType.DMA,
            pltpu.SemaphoreType.DMA,
            pltpu.SemaphoreType.DMA,
        ],
    )
    def k(h_hbm, pk_hbm, w_hbm, out_hbm,
          pb, rows, mi, wv, tag, zv, acc, msem, gsem, ssem):
        c = lax.axis_index("c")
        s = lax.axis_index("s")
        wid = c * 16 + s
        first = wid * NCH

        pltpu.sync_copy(w_hbm, wv)

        @pl.loop(0, ZR)
        def _(r):
            for j in range(H_DIM // 16):
                zv[r, pl.ds(j * 16, 16)] = jnp.zeros((16,), jnp.float32)

        @pl.loop(0, ROWS_PER_SUB // ZR)
        def _(z):
            pltpu.sync_copy(
                zv, acc.at[pl.ds(s * ROWS_PER_SUB + z * ZR, ZR), :])

        plsc.subcore_barrier()

        lane = jnp.arange(16, dtype=jnp.int32)

        pltpu.async_copy(pk_hbm.at[pl.ds(first * PKW, PKW)],
                         pb.at[pl.ds(0, PKW)], msem).wait()
        pltpu.async_copy(h_hbm.at[pb.at[pl.ds(0, CHUNK)]],
                         rows.at[0], gsem)
        pltpu.async_copy(pk_hbm.at[pl.ds((first + 1) * PKW, PKW)],
                         pb.at[pl.ds(PKW, PKW)], msem).wait()
        pltpu.async_copy(h_hbm.at[pb.at[pl.ds(PKW, CHUNK)]],
                         rows.at[1], gsem)
        pltpu.async_copy(pk_hbm.at[pl.ds((first + 2) * PKW, PKW)],
                         pb.at[pl.ds(2 * PKW, PKW)], msem)

        @pl.loop(0, NCH)
        def _(i):
            b3 = lax.rem(i, 3)
            b3nn = lax.rem(i + 2, 3)
            b2 = lax.rem(i, 2)
            b2n = lax.rem(i + 1, 2)

            pltpu.make_async_copy(h_hbm.at[pb.at[pl.ds(b3 * PKW, CHUNK)]],
                                  rows.at[b3], gsem).wait()

            idbase = i * CHUNK

            @pl.loop(0, G)
            def _(g):
                et16 = pb[pl.ds(b3 * PKW + 2 * CHUNK + g * 16, 16)]
                nb16 = pb[pl.ds(b3 * PKW + 3 * CHUNK + g * 16, 16)]
                nm16 = plsc.bitcast(nb16, jnp.float32)
                def _edge_loads(el):
                    rel = et16[el]
                    r = g * 16 + el
                    wvals = []
                    for j in range(H_DIM // 32):
                        v = wv[pl.ds(rel * (H_DIM // 2) + j * 16, 16)]
                        wlo = plsc.bitcast(v << 16, jnp.float32)
                        whi = plsc.bitcast(v & jnp.int32(-65536), jnp.float32)
                        wvals += [wlo, whi]
                    rvals = [rows[b3, r, pl.ds(j * 16, 16)]
                             for j in range(H_DIM // 16)]
                    return wvals, rvals

                cur = _edge_loads(0)
                for el in range(16):
                    nxt = _edge_loads(el + 1) if el < 15 else None
                    sc = nm16[el]
                    r = g * 16 + el
                    wvals, rvals = cur
                    for j in range(H_DIM // 16):
                        rows[b3, r, pl.ds(j * 16, 16)] = (
                            rvals[j] * (wvals[j] * sc))
                    cur = nxt
                dst16 = pb[pl.ds(b3 * PKW + CHUNK + g * 16, 16)]
                plsc.store_scatter(tag, [dst16], idbase + g * 16 + lane)

            @pl.loop(0, G)
            def _(g):
                dst16 = pb[pl.ds(b3 * PKW + CHUNK + g * 16, 16)]
                ids16 = idbase + g * 16 + lane
                rb16 = plsc.load_gather(tag, [dst16])
                losei = jnp.logical_and(rb16 != ids16,
                                        rb16 >= idbase).astype(jnp.int32)
                mi[b2, 0, pl.ds(g * 16, 16)] = jnp.where(
                    losei == 1, N_NODES + lane, dst16)

                @pl.when(jnp.sum(losei) > 0)
                def _():
                    for el in range(16):
                        @pl.when(losei[el] == 1)
                        def _():
                            w = rb16[el] - idbase
                            r = g * 16 + el
                            for j in range(H_DIM // 16):
                                sl = pl.ds(j * 16, 16)
                                rows[b3, w, sl] = (rows[b3, w, sl]
                                                   + rows[b3, r, sl])

            pltpu.async_copy(rows.at[b3], acc.at[mi.at[b2, 0]], ssem,
                             add=True)

            @pl.when(i + 3 < NCH)
            def _():
                pltpu.async_copy(
                    pk_hbm.at[pl.ds((first + i + 3) * PKW, PKW)],
                    pb.at[pl.ds(b3 * PKW, PKW)], msem)

            @pl.when(i >= 1)
            def _():
                pltpu.make_async_copy(
                    rows.at[b3nn], acc.at[mi.at[b2n, 0]], ssem).wait()

            @pl.when(i + 2 < NCH)
            def _():
                pltpu.make_async_copy(
                    pk_hbm.at[pl.ds((first + i + 2) * PKW, PKW)],
                    pb.at[pl.ds(b3nn * PKW, PKW)], msem).wait()
                pltpu.async_copy(h_hbm.at[pb.at[pl.ds(b3nn * PKW, CHUNK)]],
                                 rows.at[b3nn], gsem)

        pltpu.make_async_copy(rows.at[0], acc.at[mi.at[0, 0]], ssem).wait()

        plsc.subcore_barrier()

        pltpu.sync_copy(
            acc.at[pl.ds(s * ROWS_PER_SUB, ROWS_PER_SUB), :],
            out_hbm.at[c, pl.ds(s * ROWS_PER_SUB, ROWS_PER_SUB), :])

    return k(h, pk, W)


def _combine(p, h, loop_w, b, relu):
    R = 1000

    def body(p_ref, h_ref, w_ref, b_ref, o_ref):
        acc = (p_ref[0] + p_ref[1]
               + jnp.dot(h_ref[...], w_ref[...],
                         preferred_element_type=jnp.float32)
               + b_ref[...])
        if relu:
            acc = jnp.maximum(acc, 0.0)
        o_ref[...] = acc

    return pl.pallas_call(
        body,
        grid=(N_NODES // R,),
        in_specs=[
            pl.BlockSpec((2, R, H_DIM), lambda i: (0, i, 0)),
            pl.BlockSpec((R, H_DIM), lambda i: (i, 0)),
            pl.BlockSpec((H_DIM, H_DIM), lambda i: (0, 0)),
            pl.BlockSpec((1, H_DIM), lambda i: (0, 0)),
        ],
        out_specs=pl.BlockSpec((R, H_DIM), lambda i: (i, 0)),
        out_shape=jax.ShapeDtypeStruct((N_NODES, H_DIM), jnp.float32),
    )(p, h, loop_w, b.reshape(1, H_DIM))


def _pack_w(W):
    bits = lax.bitcast_convert_type(W.astype(jnp.bfloat16),
                                    jnp.uint16).astype(jnp.uint32)
    br = bits.reshape(NUM_RELS, H_DIM // 32, 2, 16)
    packed = (br[:, :, 1, :] << 16) | br[:, :, 0, :]
    return packed.reshape(-1).astype(jnp.int32)


def kernel(x, edge_index, etype, norm, W1, loop1, b1, W2, loop2, b2):
    src = edge_index[0].astype(jnp.int32)
    dst = edge_index[1].astype(jnp.int32)
    et = etype.astype(jnp.int32)
    nbits = lax.bitcast_convert_type(norm.reshape(-1), jnp.int32)
    pk = jnp.stack([src.reshape(-1, CHUNK), dst.reshape(-1, CHUNK),
                    et.reshape(-1, CHUNK), nbits.reshape(-1, CHUNK)],
                   axis=1).reshape(-1)

    p1 = _sc_layer(x, pk, _pack_w(W1))
    h1 = _combine(p1, x, loop1, b1, relu=True)
    p2 = _sc_layer(h1, pk, _pack_w(W2))
    return _combine(p2, h1, loop2, b2, relu=False)


# speedup vs baseline: 2.0817x; 1.0071x over previous
import dataclasses
import functools

import jax
import jax.numpy as jnp
from jax import lax
from jax.experimental import pallas as pl
from jax.experimental.pallas import tpu as pltpu
from jax.experimental.pallas import tpu_sc as plsc

N_NODES = 10000
H_DIM = 128
N_EDGES = 320000
NUM_RELS = 16

NW = 32
CHUNK = 80
G = CHUNK // 16
NCH = N_EDGES // CHUNK // NW
PKW = 4 * CHUNK
NPAD = 10240
ROWS_PER_SUB = NPAD // 16
ZR = 32


def _sc_layer(h, pk, W):
    mesh = plsc.VectorSubcoreMesh(core_axis_name="c", subcore_axis_name="s")
    cp = pltpu.CompilerParams()
    if "needs_layout_passes" in pltpu.CompilerParams.__dataclass_fields__:
        cp = dataclasses.replace(cp, needs_layout_passes=False)

    @functools.partial(
        pl.kernel,
        compiler_params=cp,
        out_type=jax.ShapeDtypeStruct((2, NPAD, H_DIM), jnp.float32),
        mesh=mesh,
        scratch_types=[
            pltpu.VMEM((3 * PKW,), jnp.int32),
            pltpu.VMEM((3, CHUNK, H_DIM), jnp.float32),
            pltpu.VMEM((2, 1, CHUNK), jnp.int32),
            pltpu.VMEM((NUM_RELS * H_DIM // 2,), jnp.int32),
            pltpu.VMEM((NPAD,), jnp.int32),
            pltpu.VMEM((ZR, H_DIM), jnp.float32),
            pltpu.VMEM_SHARED((NPAD, H_DIM), jnp.float32),
            pltpu.SemaphoreType.DMA,
            pltpu.SemaphoreType.DMA,
            pltpu.SemaphoreType.DMA,
        ],
    )
    def k(h_hbm, pk_hbm, w_hbm, out_hbm,
          pb, rows, mi, wv, tag, zv, acc, msem, gsem, ssem):
        c = lax.axis_index("c")
        s = lax.axis_index("s")
        wid = c * 16 + s
        first = wid * NCH

        pltpu.async_copy(pk_hbm.at[pl.ds(first * PKW, PKW)],
                         pb.at[pl.ds(0, PKW)], msem)
        wcp = pltpu.async_copy(w_hbm, wv, ssem)

        @pl.loop(0, ZR)
        def _(r):
            for j in range(H_DIM // 16):
                zv[r, pl.ds(j * 16, 16)] = jnp.zeros((16,), jnp.float32)

        pltpu.make_async_copy(pk_hbm.at[pl.ds(first * PKW, PKW)],
                              pb.at[pl.ds(0, PKW)], msem).wait()
        pltpu.async_copy(h_hbm.at[pb.at[pl.ds(0, CHUNK)]],
                         rows.at[0], gsem)
        pltpu.async_copy(pk_hbm.at[pl.ds((first + 1) * PKW, PKW)],
                         pb.at[pl.ds(PKW, PKW)], msem).wait()
        pltpu.async_copy(h_hbm.at[pb.at[pl.ds(PKW, CHUNK)]],
                         rows.at[1], gsem)
        pltpu.async_copy(pk_hbm.at[pl.ds((first + 2) * PKW, PKW)],
                         pb.at[pl.ds(2 * PKW, PKW)], msem)

        @pl.loop(0, ROWS_PER_SUB // ZR)
        def _(z):
            pltpu.sync_copy(
                zv, acc.at[pl.ds(s * ROWS_PER_SUB + z * ZR, ZR), :])
        wcp.wait()

        plsc.subcore_barrier()

        lane = jnp.arange(16, dtype=jnp.int32)

        @pl.loop(0, NCH)
        def _(i):
            b3 = lax.rem(i, 3)
            b3nn = lax.rem(i + 2, 3)
            b2 = lax.rem(i, 2)
            b2n = lax.rem(i + 1, 2)

            pltpu.make_async_copy(h_hbm.at[pb.at[pl.ds(b3 * PKW, CHUNK)]],
                                  rows.at[b3], gsem).wait()

            idbase = i * CHUNK

            @pl.loop(0, G)
            def _(g):
                et16 = pb[pl.ds(b3 * PKW + 2 * CHUNK + g * 16, 16)]
                nb16 = pb[pl.ds(b3 * PKW + 3 * CHUNK + g * 16, 16)]
                nm16 = plsc.bitcast(nb16, jnp.float32)
                def _edge_loads(el):
                    rel = et16[el]
                    r = g * 16 + el
                    wvals = []
                    for j in range(H_DIM // 32):
                        v = wv[pl.ds(rel * (H_DIM // 2) + j * 16, 16)]
                        wlo = plsc.bitcast(v << 16, jnp.float32)
                        whi = plsc.bitcast(v & jnp.int32(-65536), jnp.float32)
                        wvals += [wlo, whi]
                    rvals = [rows[b3, r, pl.ds(j * 16, 16)]
                             for j in range(H_DIM // 16)]
                    return wvals, rvals

                cur = _edge_loads(0)
                for el in range(16):
                    nxt = _edge_loads(el + 1) if el < 15 else None
                    sc = nm16[el]
                    r = g * 16 + el
                    wvals, rvals = cur
                    for j in range(H_DIM // 16):
                        rows[b3, r, pl.ds(j * 16, 16)] = (
                            rvals[j] * (wvals[j] * sc))
                    cur = nxt
                dst16 = pb[pl.ds(b3 * PKW + CHUNK + g * 16, 16)]
                plsc.store_scatter(tag, [dst16], idbase + g * 16 + lane)

            @pl.loop(0, G)
            def _(g):
                dst16 = pb[pl.ds(b3 * PKW + CHUNK + g * 16, 16)]
                ids16 = idbase + g * 16 + lane
                rb16 = plsc.load_gather(tag, [dst16])
                losei = (rb16 != ids16).astype(jnp.int32)
                mi[b2, 0, pl.ds(g * 16, 16)] = jnp.where(
                    losei == 1, N_NODES + lane, dst16)

                @pl.when(jnp.sum(losei) > 0)
                def _():
                    for el in range(16):
                        @pl.when(losei[el] == 1)
                        def _():
                            w = rb16[el] - idbase
                            r = g * 16 + el
                            for j in range(H_DIM // 16):
                                sl = pl.ds(j * 16, 16)
                                rows[b3, w, sl] = (rows[b3, w, sl]
                                                   + rows[b3, r, sl])

            pltpu.async_copy(rows.at[b3], acc.at[mi.at[b2, 0]], ssem,
                             add=True)

            @pl.when(i + 3 < NCH)
            def _():
                pltpu.async_copy(
                    pk_hbm.at[pl.ds((first + i + 3) * PKW, PKW)],
                    pb.at[pl.ds(b3 * PKW, PKW)], msem)

            @pl.when(i >= 1)
            def _():
                pltpu.make_async_copy(
                    rows.at[b3nn], acc.at[mi.at[b2n, 0]], ssem).wait()

            @pl.when(i + 2 < NCH)
            def _():
                pltpu.make_async_copy(
                    pk_hbm.at[pl.ds((first + i + 2) * PKW, PKW)],
                    pb.at[pl.ds(b3nn * PKW, PKW)], msem).wait()
                pltpu.async_copy(h_hbm.at[pb.at[pl.ds(b3nn * PKW, CHUNK)]],
                                 rows.at[b3nn], gsem)

        pltpu.make_async_copy(rows.at[0], acc.at[mi.at[0, 0]], ssem).wait()

        plsc.subcore_barrier()

        pltpu.sync_copy(
            acc.at[pl.ds(s * ROWS_PER_SUB, ROWS_PER_SUB), :],
            out_hbm.at[c, pl.ds(s * ROWS_PER_SUB, ROWS_PER_SUB), :])

    return k(h, pk, W)


def _combine(p, h, loop_w, b, relu):
    R = 1000

    def body(p_ref, h_ref, w_ref, b_ref, o_ref):
        acc = (p_ref[0] + p_ref[1]
               + jnp.dot(h_ref[...], w_ref[...],
                         preferred_element_type=jnp.float32)
               + b_ref[...])
        if relu:
            acc = jnp.maximum(acc, 0.0)
        o_ref[...] = acc

    return pl.pallas_call(
        body,
        grid=(N_NODES // R,),
        in_specs=[
            pl.BlockSpec((2, R, H_DIM), lambda i: (0, i, 0)),
            pl.BlockSpec((R, H_DIM), lambda i: (i, 0)),
            pl.BlockSpec((H_DIM, H_DIM), lambda i: (0, 0)),
            pl.BlockSpec((1, H_DIM), lambda i: (0, 0)),
        ],
        out_specs=pl.BlockSpec((R, H_DIM), lambda i: (i, 0)),
        out_shape=jax.ShapeDtypeStruct((N_NODES, H_DIM), jnp.float32),
    )(p, h, loop_w, b.reshape(1, H_DIM))


def _pack_w(W):
    bits = lax.bitcast_convert_type(W.astype(jnp.bfloat16),
                                    jnp.uint16).astype(jnp.uint32)
    br = bits.reshape(NUM_RELS, H_DIM // 32, 2, 16)
    packed = (br[:, :, 1, :] << 16) | br[:, :, 0, :]
    return packed.reshape(-1).astype(jnp.int32)


def kernel(x, edge_index, etype, norm, W1, loop1, b1, W2, loop2, b2):
    src = edge_index[0].astype(jnp.int32)
    dst = edge_index[1].astype(jnp.int32)
    et = etype.astype(jnp.int32)
    nbits = lax.bitcast_convert_type(norm.reshape(-1), jnp.int32)
    pk = jnp.stack([src.reshape(-1, CHUNK), dst.reshape(-1, CHUNK),
                    et.reshape(-1, CHUNK), nbits.reshape(-1, CHUNK)],
                   axis=1).reshape(-1)

    p1 = _sc_layer(x, pk, _pack_w(W1))
    h1 = _combine(p1, x, loop1, b1, relu=True)
    p2 = _sc_layer(h1, pk, _pack_w(W2))
    return _combine(p2, h1, loop2, b2, relu=False)
